# baseline (speedup 1.0000x reference)
import functools

import jax
import jax.numpy as jnp
from jax import lax
from jax.experimental import pallas as pl
from jax.experimental.pallas import tpu as pltpu
from jax.experimental.pallas import tpu_sc as plsc

BATCH = 16384
D = 64
NUM_BANDS = 1000000
_BW = 16384
_HB = _BW // 2
_G = (NUM_BANDS + _BW - 1) // _BW
_P = _G * _HB
_NC = 2
_NS = 16
_NW = _NC * _NS
_B_PER_W = BATCH // _NW
_C = 128
_NCHUNK = _B_PER_W // _C


def _pack_body(x_ref, out_ref):
    xt = x_ref[...].T
    y = xt.astype(jnp.bfloat16)
    lo = jax.lax.bitcast_convert_type(y[:, :32], jnp.uint16).astype(jnp.uint32)
    hi = jax.lax.bitcast_convert_type(y[:, 32:], jnp.uint16).astype(jnp.uint32)
    w = ((hi << 16) | lo).astype(jnp.int32)
    q = _BW // 4
    out_ref[...] = jnp.concatenate(
        [w[:q], w[q:2 * q], w[2 * q:3 * q], w[3 * q:]], axis=1)


def _tc_pack(table_t):
    return pl.pallas_call(
        _pack_body,
        grid=(_G,),
        in_specs=[pl.BlockSpec((D, _BW), lambda i: (0, i))],
        out_specs=pl.BlockSpec((_BW // 4, 2 * D), lambda i: (i, 0)),
        out_shape=jax.ShapeDtypeStruct((_G * (_BW // 4), 2 * D), jnp.int32),
    )(table_t)


def _sc_pair_gather(table2, prow):
    mesh = plsc.VectorSubcoreMesh(core_axis_name="c", subcore_axis_name="s")

    @functools.partial(
        pl.kernel,
        mesh=mesh,
        out_type=jax.ShapeDtypeStruct((BATCH, 2 * D), jnp.int32),
        scratch_types=[
            pltpu.VMEM((_NCHUNK, _C), jnp.int32),
            pltpu.VMEM((_B_PER_W, 2 * D), jnp.int32),
            pltpu.SemaphoreType.DMA,
        ],
    )
    def k(table_hbm, idx_hbm, out_hbm, idx_v, rows_v, sem):
        wid = lax.axis_index("s") * _NC + lax.axis_index("c")
        base = wid * _B_PER_W
        for c in range(_NCHUNK):
            pltpu.sync_copy(idx_hbm.at[pl.ds(base + c * _C, _C)], idx_v.at[c])
        copies = []
        for c in range(_NCHUNK):
            copies.append(pltpu.async_copy(
                table_hbm.at[idx_v.at[c]],
                rows_v.at[pl.ds(c * _C, _C)], sem))
        for cp in copies:
            cp.wait()
        pltpu.sync_copy(rows_v, out_hbm.at[pl.ds(base, _B_PER_W)])

    return k(table2, prow)


def _dense_body(g_ref, m0_ref, m1_ref, gamma_ref, beta_ref, w1_ref, b1_ref,
                w2_ref, b2_ref, out_ref):
    g = g_ref[...]
    m0 = m0_ref[...] > 0
    m1 = m1_ref[...] > 0
    a = jnp.where(m0, g[:, 32:64], g[:, :32])
    b = jnp.where(m0, g[:, 96:128], g[:, 64:96])
    w = jnp.where(m1, b, a)
    lo = jax.lax.bitcast_convert_type(
        ((w & 0xFFFF) << 16).astype(jnp.int32), jnp.float32)
    hi = jax.lax.bitcast_convert_type(
        (w & jnp.int32(-65536)).astype(jnp.int32), jnp.float32)
    x = jnp.concatenate([lo, hi], axis=1)
    mu = jnp.mean(x, axis=1, keepdims=True)
    var = jnp.mean((x - mu) ** 2, axis=1, keepdims=True)
    h = (x - mu) * lax.rsqrt(var + 1e-5) * gamma_ref[...] + beta_ref[...]
    h = jnp.dot(h, w1_ref[...], preferred_element_type=jnp.float32) + b1_ref[...]
    h = h * jax.nn.sigmoid(h)
    h = jnp.dot(h, w2_ref[...], preferred_element_type=jnp.float32) + b2_ref[...]
    out_ref[...] = h


def _tc_dense(g, m0, m1, gamma, beta, W1, b1, W2, b2):
    blk = 2048
    grid = (BATCH // blk,)
    param = pl.BlockSpec((1, D), lambda i: (0, 0))
    wspec = pl.BlockSpec((D, D), lambda i: (0, 0))
    return pl.pallas_call(
        _dense_body,
        grid=grid,
        in_specs=[
            pl.BlockSpec((blk, 2 * D), lambda i: (i, 0)),
            pl.BlockSpec((blk, 1), lambda i: (i, 0)),
            pl.BlockSpec((blk, 1), lambda i: (i, 0)),
            param, param, wspec, param, wspec, param,
        ],
        out_specs=pl.BlockSpec((blk, D), lambda i: (i, 0)),
        out_shape=jax.ShapeDtypeStruct((BATCH, D), jnp.float32),
    )(g, m0.reshape(BATCH, 1), m1.reshape(BATCH, 1), gamma.reshape(1, D),
      beta.reshape(1, D), W1, b1.reshape(1, D), W2, b2.reshape(1, D))


@jax.jit
def kernel(bands, band_emb, gamma, beta, W1, b1, W2, b2):
    bands = bands.astype(jnp.int32)
    table2 = _tc_pack(band_emb.T)
    chunk = bands // _BW
    r = bands % _BW
    q = _BW // 4
    prow = chunk * q + (r % q)
    qsel = r // q
    m0 = (qsel & 1).astype(jnp.int32)
    m1 = (qsel >> 1).astype(jnp.int32)
    g = _sc_pair_gather(table2, prow)
    return _tc_dense(g, m0, m1, gamma, beta, W1, b1, W2, b2)


# speedup vs baseline: 1.3250x; 1.3250x over previous
import functools

import jax
import jax.numpy as jnp
from jax import lax
from jax.experimental import pallas as pl
from jax.experimental.pallas import tpu as pltpu
from jax.experimental.pallas import tpu_sc as plsc

BATCH = 16384
D = 64
NUM_BANDS = 1000000
_BW = 16384
_HB = _BW // 2
_G = (NUM_BANDS + _BW - 1) // _BW
_P = _G * _HB
_NC = 2
_NS = 16
_NW = _NC * _NS
_B_PER_W = BATCH // _NW
_C = 128
_NCHUNK = _B_PER_W // _C


def _pack_body(x_ref, out_ref):
    xt = x_ref[...].T
    out_ref[...] = jnp.concatenate([xt[:_HB], xt[_HB:]], axis=1)


def _tc_pack(table_t):
    return pl.pallas_call(
        _pack_body,
        grid=(_G,),
        in_specs=[pl.BlockSpec((D, _BW), lambda i: (0, i))],
        out_specs=pl.BlockSpec((_HB, 2 * D), lambda i: (i, 0)),
        out_shape=jax.ShapeDtypeStruct((_P, 2 * D), jnp.float32),
    )(table_t)


def _sc_pair_gather(table2, prow):
    mesh = plsc.VectorSubcoreMesh(core_axis_name="c", subcore_axis_name="s")

    @functools.partial(
        pl.kernel,
        mesh=mesh,
        out_type=jax.ShapeDtypeStruct((BATCH, 2 * D), jnp.float32),
        scratch_types=[
            pltpu.VMEM((_NCHUNK, _C), jnp.int32),
            pltpu.VMEM((_B_PER_W, 2 * D), jnp.float32),
            pltpu.SemaphoreType.DMA,
        ],
    )
    def k(table_hbm, idx_hbm, out_hbm, idx_v, rows_v, sem):
        wid = lax.axis_index("s") * _NC + lax.axis_index("c")
        base = wid * _B_PER_W
        for c in range(_NCHUNK):
            pltpu.sync_copy(idx_hbm.at[pl.ds(base + c * _C, _C)], idx_v.at[c])
        copies = []
        for c in range(_NCHUNK):
            copies.append(pltpu.async_copy(
                table_hbm.at[idx_v.at[c]],
                rows_v.at[pl.ds(c * _C, _C)], sem))
        for cp in copies:
            cp.wait()
        pltpu.sync_copy(rows_v, out_hbm.at[pl.ds(base, _B_PER_W)])

    return k(table2, prow)


def _dense_body(g_ref, m_ref, gamma_ref, beta_ref, w1_ref, b1_ref, w2_ref,
                b2_ref, out_ref):
    g = g_ref[...]
    m = m_ref[...]
    x = g[:, :D] + m * (g[:, D:] - g[:, :D])
    mu = jnp.mean(x, axis=1, keepdims=True)
    var = jnp.mean((x - mu) ** 2, axis=1, keepdims=True)
    h = (x - mu) * lax.rsqrt(var + 1e-5) * gamma_ref[...] + beta_ref[...]
    h = jnp.dot(h, w1_ref[...], preferred_element_type=jnp.float32) + b1_ref[...]
    h = h * jax.nn.sigmoid(h)
    h = jnp.dot(h, w2_ref[...], preferred_element_type=jnp.float32) + b2_ref[...]
    out_ref[...] = h


def _tc_dense(g, m, gamma, beta, W1, b1, W2, b2):
    blk = 2048
    grid = (BATCH // blk,)
    param = pl.BlockSpec((1, D), lambda i: (0, 0))
    wspec = pl.BlockSpec((D, D), lambda i: (0, 0))
    return pl.pallas_call(
        _dense_body,
        grid=grid,
        in_specs=[
            pl.BlockSpec((blk, 2 * D), lambda i: (i, 0)),
            pl.BlockSpec((blk, 1), lambda i: (i, 0)),
            param, param, wspec, param, wspec, param,
        ],
        out_specs=pl.BlockSpec((blk, D), lambda i: (i, 0)),
        out_shape=jax.ShapeDtypeStruct((BATCH, D), jnp.float32),
    )(g, m.reshape(BATCH, 1), gamma.reshape(1, D), beta.reshape(1, D),
      W1, b1.reshape(1, D), W2, b2.reshape(1, D))


@jax.jit
def kernel(bands, band_emb, gamma, beta, W1, b1, W2, b2):
    bands = bands.astype(jnp.int32)
    table2 = _tc_pack(band_emb.T)
    chunk = bands // _BW
    r = bands % _BW
    prow = chunk * _HB + (r & (_HB - 1))
    m = (r >= _HB).astype(jnp.float32)
    g = _sc_pair_gather(table2, prow)
    return _tc_dense(g, m, gamma, beta, W1, b1, W2, b2)


# speedup vs baseline: 1.6598x; 1.2527x over previous
import functools

import jax
import jax.numpy as jnp
from jax import lax
from jax.experimental import pallas as pl
from jax.experimental.pallas import tpu as pltpu
from jax.experimental.pallas import tpu_sc as plsc

BATCH = 16384
D = 64
NUM_BANDS = 1000000
_BW = 16384
_Q = _BW // 4
_G = (NUM_BANDS + _BW - 1) // _BW
_P = _G * _Q
_NC = 2
_NS = 16
_NW = _NC * _NS
_B_PER_W = BATCH // _NW
_C = 128
_NCHUNK = _B_PER_W // _C


def _pack_body(x_ref, out_ref):
    xt = x_ref[...].T
    y = xt.astype(jnp.bfloat16)
    w = pltpu.bitcast(y, jnp.int32)
    out_ref[...] = jnp.concatenate([w[:_Q], w[_Q:]], axis=1)


def _tc_pack(table_t):
    return pl.pallas_call(
        _pack_body,
        grid=(_G,),
        in_specs=[pl.BlockSpec((D, _BW), lambda i: (0, i))],
        out_specs=pl.BlockSpec((_Q, 2 * D), lambda i: (i, 0)),
        out_shape=jax.ShapeDtypeStruct((_P, 2 * D), jnp.int32),
    )(table_t)


def _sc_quad_gather(table2, prow):
    mesh = plsc.VectorSubcoreMesh(core_axis_name="c", subcore_axis_name="s")

    @functools.partial(
        pl.kernel,
        mesh=mesh,
        out_type=jax.ShapeDtypeStruct((BATCH, 2 * D), jnp.int32),
        scratch_types=[
            pltpu.VMEM((_NCHUNK, _C), jnp.int32),
            pltpu.VMEM((_B_PER_W, 2 * D), jnp.int32),
            pltpu.SemaphoreType.DMA,
        ],
    )
    def k(table_hbm, idx_hbm, out_hbm, idx_v, rows_v, sem):
        wid = lax.axis_index("s") * _NC + lax.axis_index("c")
        base = wid * _B_PER_W
        for c in range(_NCHUNK):
            pltpu.sync_copy(idx_hbm.at[pl.ds(base + c * _C, _C)], idx_v.at[c])
        copies = []
        for c in range(_NCHUNK):
            copies.append(pltpu.async_copy(
                table_hbm.at[idx_v.at[c]],
                rows_v.at[pl.ds(c * _C, _C)], sem))
        for cp in copies:
            cp.wait()
        pltpu.sync_copy(rows_v, out_hbm.at[pl.ds(base, _B_PER_W)])

    return k(table2, prow)


def _dense_body(g_ref, m0_ref, m1_ref, gamma_ref, beta_ref, w1_ref, b1_ref,
                w2_ref, b2_ref, out_ref):
    g = g_ref[...]
    mm0 = m0_ref[...]
    mm1 = m1_ref[...]
    a, b = g[:, :D], g[:, D:]
    t = a ^ ((a ^ b) & mm0)
    u = jax.lax.shift_left(t, jnp.int32(16))
    v = t & jnp.int32(-65536)
    x = jax.lax.bitcast_convert_type(u ^ ((u ^ v) & mm1), jnp.float32)
    mu = jnp.mean(x, axis=1, keepdims=True)
    var = jnp.mean((x - mu) ** 2, axis=1, keepdims=True)
    h = (x - mu) * lax.rsqrt(var + 1e-5) * gamma_ref[...] + beta_ref[...]
    h = jnp.dot(h, w1_ref[...], preferred_element_type=jnp.float32) + b1_ref[...]
    h = h * jax.nn.sigmoid(h)
    h = jnp.dot(h, w2_ref[...], preferred_element_type=jnp.float32) + b2_ref[...]
    out_ref[...] = h


def _tc_dense(g, m0, m1, gamma, beta, W1, b1, W2, b2):
    blk = 2048
    grid = (BATCH // blk,)
    param = pl.BlockSpec((1, D), lambda i: (0, 0))
    wspec = pl.BlockSpec((D, D), lambda i: (0, 0))
    return pl.pallas_call(
        _dense_body,
        grid=grid,
        in_specs=[
            pl.BlockSpec((blk, 2 * D), lambda i: (i, 0)),
            pl.BlockSpec((blk, D), lambda i: (i, 0)),
            pl.BlockSpec((blk, D), lambda i: (i, 0)),
            param, param, wspec, param, wspec, param,
        ],
        out_specs=pl.BlockSpec((blk, D), lambda i: (i, 0)),
        out_shape=jax.ShapeDtypeStruct((BATCH, D), jnp.float32),
    )(g, m0, m1, gamma.reshape(1, D), beta.reshape(1, D), W1,
      b1.reshape(1, D), W2, b2.reshape(1, D))


@jax.jit
def kernel(bands, band_emb, gamma, beta, W1, b1, W2, b2):
    bands = bands.astype(jnp.int32)
    table2 = _tc_pack(band_emb.T)
    s = bands >> 1
    chunk = s // (_BW // 2)
    sr = s % (_BW // 2)
    prow = chunk * _Q + (sr % _Q)
    m0 = jnp.broadcast_to((-(sr // _Q)).astype(jnp.int32)[:, None],
                          (BATCH, D))
    m1 = jnp.broadcast_to((-(bands & 1)).astype(jnp.int32)[:, None],
                          (BATCH, D))
    g = _sc_quad_gather(table2, prow)
    return _tc_dense(g, m0, m1, gamma, beta, W1, b1, W2, b2)


# speedup vs baseline: 1.8064x; 1.0883x over previous
import functools

import jax
import jax.numpy as jnp
from jax import lax
from jax.experimental import pallas as pl
from jax.experimental.pallas import tpu as pltpu
from jax.experimental.pallas import tpu_sc as plsc

BATCH = 16384
D = 64
NUM_BANDS = 1000000
_BW = 32768
_Q = _BW // 4
_G = (NUM_BANDS + _BW - 1) // _BW
_P = _G * _Q
_NC = 2
_NS = 16
_NW = _NC * _NS
_B_PER_W = BATCH // _NW
_C = 128
_NCHUNK = _B_PER_W // _C


def _pack_body(x_ref, out_ref):
    xt = x_ref[...].T
    y = xt.astype(jnp.bfloat16)
    w = pltpu.bitcast(y, jnp.int32)
    out_ref[...] = jnp.concatenate([w[:_Q], w[_Q:]], axis=1)


def _tc_pack(table_t):
    return pl.pallas_call(
        _pack_body,
        grid=(_G,),
        in_specs=[pl.BlockSpec((D, _BW), lambda i: (0, i))],
        out_specs=pl.BlockSpec((_Q, 2 * D), lambda i: (i, 0)),
        out_shape=jax.ShapeDtypeStruct((_P, 2 * D), jnp.int32),
    )(table_t)


def _sc_quad_gather(table2, prow):
    mesh = plsc.VectorSubcoreMesh(core_axis_name="c", subcore_axis_name="s")

    @functools.partial(
        pl.kernel,
        mesh=mesh,
        out_type=jax.ShapeDtypeStruct((BATCH, 2 * D), jnp.int32),
        scratch_types=[
            pltpu.VMEM((_NCHUNK, _C), jnp.int32),
            pltpu.VMEM((_B_PER_W, 2 * D), jnp.int32),
            pltpu.SemaphoreType.DMA,
        ],
    )
    def k(table_hbm, idx_hbm, out_hbm, idx_v, rows_v, sem):
        wid = lax.axis_index("s") * _NC + lax.axis_index("c")
        base = wid * _B_PER_W
        for c in range(_NCHUNK):
            pltpu.sync_copy(idx_hbm.at[pl.ds(base + c * _C, _C)], idx_v.at[c])
        copies = []
        for c in range(_NCHUNK):
            copies.append(pltpu.async_copy(
                table_hbm.at[idx_v.at[c]],
                rows_v.at[pl.ds(c * _C, _C)], sem))
        for cp in copies:
            cp.wait()
        pltpu.sync_copy(rows_v, out_hbm.at[pl.ds(base, _B_PER_W)])

    return k(table2, prow)


def _dense_body(g_ref, m0_ref, m1_ref, gamma_ref, beta_ref, w1_ref, b1_ref,
                w2_ref, b2_ref, out_ref):
    g = g_ref[...]
    mm0 = m0_ref[...]
    mm1 = m1_ref[...]
    a, b = g[:, :D], g[:, D:]
    t = a ^ ((a ^ b) & mm0)
    u = jax.lax.shift_left(t, jnp.int32(16))
    v = t & jnp.int32(-65536)
    x = jax.lax.bitcast_convert_type(u ^ ((u ^ v) & mm1), jnp.float32)
    mu = jnp.mean(x, axis=1, keepdims=True)
    var = jnp.mean((x - mu) ** 2, axis=1, keepdims=True)
    h = (x - mu) * lax.rsqrt(var + 1e-5) * gamma_ref[...] + beta_ref[...]
    h = jnp.dot(h, w1_ref[...], preferred_element_type=jnp.float32) + b1_ref[...]
    h = h * jax.nn.sigmoid(h)
    h = jnp.dot(h, w2_ref[...], preferred_element_type=jnp.float32) + b2_ref[...]
    out_ref[...] = h


def _tc_dense(g, m0, m1, gamma, beta, W1, b1, W2, b2):
    blk = 2048
    grid = (BATCH // blk,)
    param = pl.BlockSpec((1, D), lambda i: (0, 0))
    wspec = pl.BlockSpec((D, D), lambda i: (0, 0))
    return pl.pallas_call(
        _dense_body,
        grid=grid,
        in_specs=[
            pl.BlockSpec((blk, 2 * D), lambda i: (i, 0)),
            pl.BlockSpec((blk, D), lambda i: (i, 0)),
            pl.BlockSpec((blk, D), lambda i: (i, 0)),
            param, param, wspec, param, wspec, param,
        ],
        out_specs=pl.BlockSpec((blk, D), lambda i: (i, 0)),
        out_shape=jax.ShapeDtypeStruct((BATCH, D), jnp.float32),
    )(g, m0, m1, gamma.reshape(1, D), beta.reshape(1, D), W1,
      b1.reshape(1, D), W2, b2.reshape(1, D))


@jax.jit
def kernel(bands, band_emb, gamma, beta, W1, b1, W2, b2):
    bands = bands.astype(jnp.int32)
    table2 = _tc_pack(band_emb.T)
    s = bands >> 1
    chunk = s // (_BW // 2)
    sr = s % (_BW // 2)
    prow = chunk * _Q + (sr % _Q)
    m0 = jnp.broadcast_to((-(sr // _Q)).astype(jnp.int32)[:, None],
                          (BATCH, D))
    m1 = jnp.broadcast_to((-(bands & 1)).astype(jnp.int32)[:, None],
                          (BATCH, D))
    g = _sc_quad_gather(table2, prow)
    return _tc_dense(g, m0, m1, gamma, beta, W1, b1, W2, b2)


# speedup vs baseline: 1.8205x; 1.0078x over previous
import functools

import jax
import jax.numpy as jnp
from jax import lax
from jax.experimental import pallas as pl
from jax.experimental.pallas import tpu as pltpu
from jax.experimental.pallas import tpu_sc as plsc

BATCH = 16384
D = 64
NUM_BANDS = 1000000
_BW = 49152
_Q = _BW // 4
_G = (NUM_BANDS + _BW - 1) // _BW
_P = _G * _Q
_NC = 2
_NS = 16
_NW = _NC * _NS
_B_PER_W = BATCH // _NW
_C = 128
_NCHUNK = _B_PER_W // _C


def _pack_body(x_ref, out_ref):
    xt = x_ref[...].T
    y = xt.astype(jnp.bfloat16)
    w = pltpu.bitcast(y, jnp.int32)
    out_ref[...] = jnp.concatenate([w[:_Q], w[_Q:]], axis=1)


def _tc_pack(table_t):
    return pl.pallas_call(
        _pack_body,
        grid=(_G,),
        in_specs=[pl.BlockSpec((D, _BW), lambda i: (0, i))],
        out_specs=pl.BlockSpec((_Q, 2 * D), lambda i: (i, 0)),
        out_shape=jax.ShapeDtypeStruct((_P, 2 * D), jnp.int32),
    )(table_t)


def _sc_quad_gather(table2, prow):
    mesh = plsc.VectorSubcoreMesh(core_axis_name="c", subcore_axis_name="s")

    @functools.partial(
        pl.kernel,
        mesh=mesh,
        out_type=jax.ShapeDtypeStruct((BATCH, 2 * D), jnp.int32),
        scratch_types=[
            pltpu.VMEM((_NCHUNK, _C), jnp.int32),
            pltpu.VMEM((_B_PER_W, 2 * D), jnp.int32),
            pltpu.SemaphoreType.DMA,
        ],
    )
    def k(table_hbm, idx_hbm, out_hbm, idx_v, rows_v, sem):
        wid = lax.axis_index("s") * _NC + lax.axis_index("c")
        base = wid * _B_PER_W
        for c in range(_NCHUNK):
            pltpu.sync_copy(idx_hbm.at[pl.ds(base + c * _C, _C)], idx_v.at[c])
        copies = []
        for c in range(_NCHUNK):
            copies.append(pltpu.async_copy(
                table_hbm.at[idx_v.at[c]],
                rows_v.at[pl.ds(c * _C, _C)], sem))
        for cp in copies:
            cp.wait()
        pltpu.sync_copy(rows_v, out_hbm.at[pl.ds(base, _B_PER_W)])

    return k(table2, prow)


def _dense_body(g_ref, m0_ref, m1_ref, gamma_ref, beta_ref, w1_ref, b1_ref,
                w2_ref, b2_ref, out_ref):
    g = g_ref[...]
    mm0 = m0_ref[...]
    mm1 = m1_ref[...]
    a, b = g[:, :D], g[:, D:]
    t = a ^ ((a ^ b) & mm0)
    u = jax.lax.shift_left(t, jnp.int32(16))
    v = t & jnp.int32(-65536)
    x = jax.lax.bitcast_convert_type(u ^ ((u ^ v) & mm1), jnp.float32)
    mu = jnp.mean(x, axis=1, keepdims=True)
    var = jnp.mean((x - mu) ** 2, axis=1, keepdims=True)
    h = (x - mu) * lax.rsqrt(var + 1e-5) * gamma_ref[...] + beta_ref[...]
    h = jnp.dot(h, w1_ref[...], preferred_element_type=jnp.float32) + b1_ref[...]
    h = h * jax.nn.sigmoid(h)
    h = jnp.dot(h, w2_ref[...], preferred_element_type=jnp.float32) + b2_ref[...]
    out_ref[...] = h


def _tc_dense(g, m0, m1, gamma, beta, W1, b1, W2, b2):
    blk = 2048
    grid = (BATCH // blk,)
    param = pl.BlockSpec((1, D), lambda i: (0, 0))
    wspec = pl.BlockSpec((D, D), lambda i: (0, 0))
    return pl.pallas_call(
        _dense_body,
        grid=grid,
        in_specs=[
            pl.BlockSpec((blk, 2 * D), lambda i: (i, 0)),
            pl.BlockSpec((blk, D), lambda i: (i, 0)),
            pl.BlockSpec((blk, D), lambda i: (i, 0)),
            param, param, wspec, param, wspec, param,
        ],
        out_specs=pl.BlockSpec((blk, D), lambda i: (i, 0)),
        out_shape=jax.ShapeDtypeStruct((BATCH, D), jnp.float32),
    )(g, m0, m1, gamma.reshape(1, D), beta.reshape(1, D), W1,
      b1.reshape(1, D), W2, b2.reshape(1, D))


@jax.jit
def kernel(bands, band_emb, gamma, beta, W1, b1, W2, b2):
    bands = bands.astype(jnp.int32)
    table2 = _tc_pack(band_emb.T)
    s = bands >> 1
    chunk = s // (_BW // 2)
    sr = s % (_BW // 2)
    prow = chunk * _Q + (sr % _Q)
    m0 = jnp.broadcast_to((-(sr // _Q)).astype(jnp.int32)[:, None],
                          (BATCH, D))
    m1 = jnp.broadcast_to((-(bands & 1)).astype(jnp.int32)[:, None],
                          (BATCH, D))
    g = _sc_quad_gather(table2, prow)
    return _tc_dense(g, m0, m1, gamma, beta, W1, b1, W2, b2)
